# baseline (speedup 1.0000x reference)
import functools

import jax
import jax.numpy as jnp
from jax import lax
from jax.experimental import pallas as pl
from jax.experimental.pallas import tpu as pltpu
from jax.experimental.pallas import tpu_sc as plsc

N = 10000
D = 128
E = 160000
NE = 8
BN = float(1.0 / (1.0 + 1e-5) ** 0.5)
NC = 2
NS = 16
NW = NC * NS
NP = 10240
EP = 163840
CH = 128
DNCH = EP // NW // CH
PNCH = EP // NS // CH
RPT = NP // NS
DH = D // NC

BLK = 1000
NB = N // BLK


def _sc_mesh():
    return plsc.VectorSubcoreMesh(core_axis_name="c", subcore_axis_name="s")


def _deg_kernel(dstp_hbm, zrow_hbm, out_hbm, idx_v, ones_v, shared,
                sem0, sem1):
    cid = lax.axis_index("c")
    sid = lax.axis_index("s")
    wid = sid * NC + cid

    def _ones(i, carry):
        ones_v[pl.ds(i * 16, 16)] = jnp.ones((16,), jnp.float32)
        return carry

    lax.fori_loop(0, CH // 16, _ones, 0)

    row0 = pl.multiple_of(sid * RPT, 8)
    pltpu.sync_copy(zrow_hbm, shared.at[pl.ds(row0, RPT)])
    pltpu.sync_copy(dstp_hbm.at[wid], idx_v)
    plsc.subcore_barrier()

    sems = [sem0, sem1]
    pltpu.async_copy(ones_v, shared.at[idx_v.at[0]], sem0, add=True)
    pltpu.async_copy(ones_v, shared.at[idx_v.at[1]], sem1, add=True)

    def _body(g, carry):
        for b in range(2):
            k = 2 * g + b
            pltpu.make_async_copy(ones_v, shared.at[idx_v.at[k]],
                                  sems[b]).wait()

            @pl.when(k + 2 < DNCH)
            def _():
                pltpu.async_copy(ones_v, shared.at[idx_v.at[k + 2]], sems[b],
                                 add=True)
        return carry

    lax.fori_loop(0, DNCH // 2, _body, 0)
    plsc.subcore_barrier()
    pltpu.sync_copy(shared.at[pl.ds(row0, RPT)],
                    out_hbm.at[cid, pl.ds(row0, RPT)])


def _deg_call(dstp3):
    zrow = jnp.zeros((RPT,), jnp.float32)
    return pl.kernel(
        _deg_kernel,
        out_type=jax.ShapeDtypeStruct((NC, NP), jnp.float32),
        mesh=_sc_mesh(),
        scratch_types=[
            pltpu.VMEM((DNCH, CH), jnp.int32),
            pltpu.VMEM((CH,), jnp.float32),
            pltpu.VMEM_SHARED((NP,), jnp.float32),
            pltpu.SemaphoreType.DMA,
            pltpu.SemaphoreType.DMA,
        ],
    )(dstp3, zrow)


def _prop_kernel(srcp_hbm, dstp_hbm, yr_hbm, zfull_hbm, out_hbm,
                 sidx_v, didx_v, rows0_v, rows1_v, rows2_v, rows3_v, rows4_v,
                 shared, gsem0, gsem1, gsem2, gsem3, gsem4,
                 ssem0, ssem1, ssem2, ssem3, ssem4):
    cid = lax.axis_index("c")
    sid = lax.axis_index("s")

    row0 = pl.multiple_of(sid * RPT, 8)
    pltpu.sync_copy(zfull_hbm, rows0_v)
    for j in range(RPT // CH):
        pltpu.sync_copy(rows0_v,
                        shared.at[pl.ds(pl.multiple_of(row0 + j * CH, 8), CH)])
    pltpu.sync_copy(srcp_hbm.at[cid, sid], sidx_v)
    pltpu.sync_copy(dstp_hbm.at[sid], didx_v)
    plsc.subcore_barrier()

    rows = [rows0_v, rows1_v, rows2_v, rows3_v, rows4_v]
    gsems = [gsem0, gsem1, gsem2, gsem3, gsem4]
    ssems = [ssem0, ssem1, ssem2, ssem3, ssem4]

    pltpu.async_copy(yr_hbm.at[sidx_v.at[0]], rows[0], gsems[0])
    pltpu.async_copy(yr_hbm.at[sidx_v.at[1]], rows[1], gsems[1])

    def _body(g, carry):
        for b in range(5):
            k = 5 * g + b
            bn = (b + 2) % 5
            pltpu.make_async_copy(yr_hbm.at[sidx_v.at[k]], rows[b],
                                  gsems[b]).wait()
            pltpu.async_copy(rows[b], shared.at[didx_v.at[k]], ssems[b],
                             add=True)

            @pl.when(k >= 3)
            def _():
                pltpu.make_async_copy(rows[bn], shared.at[didx_v.at[k - 3]],
                                      ssems[bn]).wait()

            @pl.when(k + 2 < PNCH)
            def _():
                pltpu.async_copy(yr_hbm.at[sidx_v.at[k + 2]], rows[bn],
                                 gsems[bn])
        return carry

    lax.fori_loop(0, PNCH // 5, _body, 0)
    for kk in (PNCH - 3, PNCH - 2, PNCH - 1):
        pltpu.make_async_copy(rows[kk % 5], shared.at[didx_v.at[kk]],
                              ssems[kk % 5]).wait()
    plsc.subcore_barrier()
    nj = RPT // CH
    for j in range(nj):
        r = pl.multiple_of(row0 + j * CH, 8)
        buf = rows[j % 2]
        if j >= 2:
            rp = pl.multiple_of(row0 + (j - 2) * CH, 8)
            pltpu.make_async_copy(buf, out_hbm.at[cid, pl.ds(rp, CH)],
                                  gsems[j % 2]).wait()
        pltpu.sync_copy(shared.at[pl.ds(r, CH)], buf)
        pltpu.async_copy(buf, out_hbm.at[cid, pl.ds(r, CH)], gsems[j % 2])
    for j in range(nj - 2, nj):
        r = pl.multiple_of(row0 + j * CH, 8)
        pltpu.make_async_copy(rows[j % 2], out_hbm.at[cid, pl.ds(r, CH)],
                              gsems[j % 2]).wait()


def _prop_call(srcp4, dstp3t, yr):
    zfull = jnp.zeros((CH, DH), jnp.float32)
    return pl.kernel(
        _prop_kernel,
        out_type=jax.ShapeDtypeStruct((NC, NP, DH), jnp.float32),
        mesh=_sc_mesh(),
        compiler_params=pltpu.CompilerParams(use_tc_tiling_on_sc=False),
        scratch_types=[
            pltpu.VMEM((PNCH, CH), jnp.int32),
            pltpu.VMEM((PNCH, CH), jnp.int32),
            pltpu.VMEM((CH, DH), jnp.float32),
            pltpu.VMEM((CH, DH), jnp.float32),
            pltpu.VMEM((CH, DH), jnp.float32),
            pltpu.VMEM((CH, DH), jnp.float32),
            pltpu.VMEM((CH, DH), jnp.float32),
            pltpu.VMEM_SHARED((NP, DH), jnp.float32),
            pltpu.SemaphoreType.DMA,
            pltpu.SemaphoreType.DMA,
            pltpu.SemaphoreType.DMA,
            pltpu.SemaphoreType.DMA,
            pltpu.SemaphoreType.DMA,
            pltpu.SemaphoreType.DMA,
            pltpu.SemaphoreType.DMA,
            pltpu.SemaphoreType.DMA,
            pltpu.SemaphoreType.DMA,
            pltpu.SemaphoreType.DMA,
        ],
    )(srcp4, dstp3t, yr, zfull)


def _prep_body(x_ref, degt_ref, y_ref, dinv_ref):
    deg = 1.0 + degt_ref[:, 0:1] + degt_ref[:, 1:2]
    dinv = lax.rsqrt(deg)
    dinv_ref[...] = dinv
    y_ref[...] = x_ref[...] * dinv


def _prep_call(x, degt):
    return pl.pallas_call(
        _prep_body,
        grid=(NB,),
        in_specs=[
            pl.BlockSpec((BLK, D), lambda i: (i, 0)),
            pl.BlockSpec((BLK, 2), lambda i: (i, 0)),
        ],
        out_specs=[
            pl.BlockSpec((BLK, D), lambda i: (i, 0)),
            pl.BlockSpec((BLK, 1), lambda i: (i, 0)),
        ],
        out_shape=[
            jax.ShapeDtypeStruct((N, D), jnp.float32),
            jax.ShapeDtypeStruct((N, 1), jnp.float32),
        ],
    )(x, degt)


def _stats_body(zp_ref, y_ref, dinv_ref, wexp_ref, bexp_ref,
                z_ref, g_ref, c_ref, s_ref, t_ref):
    i = pl.program_id(0)
    by = jnp.concatenate([zp_ref[0], zp_ref[1]], axis=1)
    z = dinv_ref[...] * (by + y_ref[...])
    z_ref[...] = z
    gpart = lax.dot_general(z, z, (((0,), (0,)), ((), ())),
                            preferred_element_type=jnp.float32)
    cpart = jnp.sum(z, axis=0, keepdims=True)

    @pl.when(i == 0)
    def _init():
        g_ref[...] = gpart
        c_ref[...] = cpart
        wall = wexp_ref[...]
        ball = bexp_ref[...]
        s = jnp.zeros((D, D), jnp.float32)
        t = jnp.zeros((1, D), jnp.float32)
        for e in range(NE):
            we = wall[e]
            s = s + lax.dot_general(we, we, (((1,), (1,)), ((), ())),
                                    preferred_element_type=jnp.float32)
            t = t + lax.dot_general(ball[0, e:e + 1], we,
                                    (((1,), (1,)), ((), ())),
                                    preferred_element_type=jnp.float32)
        s_ref[...] = s
        t_ref[...] = t

    @pl.when(i > 0)
    def _acc():
        g_ref[...] += gpart
        c_ref[...] += cpart


def _stats_call(zp, y, dinv, W_exp, b_exp):
    b_exp3 = b_exp[None]
    return pl.pallas_call(
        _stats_body,
        grid=(NB,),
        in_specs=[
            pl.BlockSpec((NC, BLK, DH), lambda i: (0, i, 0)),
            pl.BlockSpec((BLK, D), lambda i: (i, 0)),
            pl.BlockSpec((BLK, 1), lambda i: (i, 0)),
            pl.BlockSpec((NE, D, D), lambda i: (0, 0, 0)),
            pl.BlockSpec((1, NE, D), lambda i: (0, 0, 0)),
        ],
        out_specs=[
            pl.BlockSpec((BLK, D), lambda i: (i, 0)),
            pl.BlockSpec((D, D), lambda i: (0, 0)),
            pl.BlockSpec((1, D), lambda i: (0, 0)),
            pl.BlockSpec((D, D), lambda i: (0, 0)),
            pl.BlockSpec((1, D), lambda i: (0, 0)),
        ],
        out_shape=[
            jax.ShapeDtypeStruct((N, D), jnp.float32),
            jax.ShapeDtypeStruct((D, D), jnp.float32),
            jax.ShapeDtypeStruct((1, D), jnp.float32),
            jax.ShapeDtypeStruct((D, D), jnp.float32),
            jax.ShapeDtypeStruct((1, D), jnp.float32),
        ],
    )(zp[:, :N], y, dinv, W_exp, b_exp3)


def _epi_body(z_ref, x_ref, wmp_ref, beta_ref, w1_ref, b1_ref, w2_ref, b2_ref,
              out_ref):
    mm = lax.dot_general(z_ref[...], wmp_ref[...], (((1,), (0,)), ((), ())),
                         preferred_element_type=jnp.float32)
    h = (x_ref[...] + mm + beta_ref[...]) * BN
    a1 = lax.dot_general(h, w1_ref[...], (((1,), (0,)), ((), ())),
                         preferred_element_type=jnp.float32)
    a1 = jnp.maximum(a1 + b1_ref[...], 0.0)
    ff = lax.dot_general(a1, w2_ref[...], (((1,), (0,)), ((), ())),
                         preferred_element_type=jnp.float32) + b2_ref[...]
    out_ref[...] = (h + ff) * BN


def _epi_call(z, x, Wmp, beta, W1, b1, W2, b2):
    return pl.pallas_call(
        _epi_body,
        grid=(NB,),
        in_specs=[
            pl.BlockSpec((BLK, D), lambda i: (i, 0)),
            pl.BlockSpec((BLK, D), lambda i: (i, 0)),
            pl.BlockSpec((D, D), lambda i: (0, 0)),
            pl.BlockSpec((1, D), lambda i: (0, 0)),
            pl.BlockSpec((D, 2 * D), lambda i: (0, 0)),
            pl.BlockSpec((1, 2 * D), lambda i: (0, 0)),
            pl.BlockSpec((2 * D, D), lambda i: (0, 0)),
            pl.BlockSpec((1, D), lambda i: (0, 0)),
        ],
        out_specs=pl.BlockSpec((BLK, D), lambda i: (i, 0)),
        out_shape=jax.ShapeDtypeStruct((N, D), jnp.float32),
    )(z, x, Wmp, beta, W1, b1, W2, b2)


def kernel(x, edge_index, W_exp, b_exp, W1, b1, W2, b2):
    src = edge_index[0]
    dst = edge_index[1]
    pad = EP - E
    srcp = jnp.concatenate([src, jnp.zeros((pad,), jnp.int32)])
    dstp = jnp.concatenate(
        [dst, N + (jnp.arange(pad, dtype=jnp.int32) % (NP - N))])

    dstp3 = dstp.reshape(NW, DNCH, CH)
    srcp_t = srcp.reshape(NS, PNCH, CH)
    srcp4 = jnp.stack([2 * srcp_t, 2 * srcp_t + 1])
    dstp3t = dstp.reshape(NS, PNCH, CH)

    degp = _deg_call(dstp3)
    degt = jnp.stack([degp[0, :N], degp[1, :N]], axis=1)
    y, dinv = _prep_call(x, degt)

    yr = y.reshape(2 * N, DH)
    zp = _prop_call(srcp4, dstp3t, yr)
    z, G, c, S, t = _stats_call(zp, y, dinv, W_exp, b_exp)

    c1 = c[0]
    t1 = t[0]
    wrs = jnp.sum(W_exp, axis=(0, 2))
    bsum = jnp.sum(b_exp)
    bsq = jnp.sum(b_exp * b_exp)
    M = float(NE * N * D)
    sum_all = BN * (jnp.dot(c1, wrs) + N * bsum)
    sumsq_all = BN * BN * (jnp.sum(G * S) + 2.0 * jnp.dot(c1, t1) + N * bsq)
    gm = sum_all / M
    gs = jnp.sqrt(jnp.maximum(sumsq_all - M * gm * gm, 0.0) / (M - 1.0))
    inv = 1.0 / (gs + 1e-8)

    Wm = jnp.mean(W_exp, axis=0)
    bm = jnp.mean(b_exp, axis=0)
    Wmp = (BN * inv) * Wm
    beta = ((BN * bm - gm) * inv)[None]

    return _epi_call(z, x, Wmp, beta, W1, b1[None], W2, b2[None])


# speedup vs baseline: 1.0152x; 1.0152x over previous
import functools

import jax
import jax.numpy as jnp
from jax import lax
from jax.experimental import pallas as pl
from jax.experimental.pallas import tpu as pltpu
from jax.experimental.pallas import tpu_sc as plsc

N = 10000
D = 128
E = 160000
NE = 8
BN = float(1.0 / (1.0 + 1e-5) ** 0.5)
NC = 2
NS = 16
NW = NC * NS
NP = 10240
EP = 163840
CH = 128
DNCH = EP // NW // CH
PNCH = EP // NS // CH
RPT = NP // NS
DH = D // NC

BLK = 1000
NB = N // BLK


def _sc_mesh():
    return plsc.VectorSubcoreMesh(core_axis_name="c", subcore_axis_name="s")


def _deg_kernel(dstp_hbm, zrow_hbm, out_hbm, idx_v, ones_v, shared,
                sem0, sem1):
    cid = lax.axis_index("c")
    sid = lax.axis_index("s")
    wid = sid * NC + cid

    def _ones(i, carry):
        ones_v[pl.ds(i * 16, 16)] = jnp.ones((16,), jnp.float32)
        return carry

    lax.fori_loop(0, CH // 16, _ones, 0)

    row0 = pl.multiple_of(sid * RPT, 8)
    pltpu.sync_copy(zrow_hbm, shared.at[pl.ds(row0, RPT)])
    pltpu.sync_copy(dstp_hbm.at[wid], idx_v)
    plsc.subcore_barrier()

    sems = [sem0, sem1]
    pltpu.async_copy(ones_v, shared.at[idx_v.at[0]], sem0, add=True)
    pltpu.async_copy(ones_v, shared.at[idx_v.at[1]], sem1, add=True)

    def _body(g, carry):
        for b in range(2):
            k = 2 * g + b
            pltpu.make_async_copy(ones_v, shared.at[idx_v.at[k]],
                                  sems[b]).wait()

            @pl.when(k + 2 < DNCH)
            def _():
                pltpu.async_copy(ones_v, shared.at[idx_v.at[k + 2]], sems[b],
                                 add=True)
        return carry

    lax.fori_loop(0, DNCH // 2, _body, 0)
    plsc.subcore_barrier()
    pltpu.sync_copy(shared.at[pl.ds(row0, RPT)],
                    out_hbm.at[cid, pl.ds(row0, RPT)])


def _deg_call(dstp3):
    zrow = jnp.zeros((RPT,), jnp.float32)
    return pl.kernel(
        _deg_kernel,
        out_type=jax.ShapeDtypeStruct((NC, NP), jnp.float32),
        mesh=_sc_mesh(),
        scratch_types=[
            pltpu.VMEM((DNCH, CH), jnp.int32),
            pltpu.VMEM((CH,), jnp.float32),
            pltpu.VMEM_SHARED((NP,), jnp.float32),
            pltpu.SemaphoreType.DMA,
            pltpu.SemaphoreType.DMA,
        ],
    )(dstp3, zrow)


def _prop_kernel(srcp_hbm, dstp_hbm, yr_hbm, zfull_hbm, out_hbm,
                 sidx_v, didx_v, rows0_v, rows1_v, rows2_v, rows3_v, rows4_v,
                 shared, gsem0, gsem1, gsem2, gsem3, gsem4,
                 ssem0, ssem1, ssem2, ssem3, ssem4):
    cid = lax.axis_index("c")
    sid = lax.axis_index("s")

    row0 = pl.multiple_of(sid * RPT, 8)
    pltpu.sync_copy(zfull_hbm, rows0_v)
    for j in range(RPT // CH):
        pltpu.sync_copy(rows0_v,
                        shared.at[pl.ds(pl.multiple_of(row0 + j * CH, 8), CH)])
    pltpu.sync_copy(srcp_hbm.at[cid, sid], sidx_v)
    pltpu.sync_copy(dstp_hbm.at[sid], didx_v)
    plsc.subcore_barrier()

    rows = [rows0_v, rows1_v, rows2_v, rows3_v, rows4_v]
    gsems = [gsem0, gsem1, gsem2, gsem3, gsem4]
    ssems = [ssem0, ssem1, ssem2, ssem3, ssem4]

    pltpu.async_copy(yr_hbm.at[sidx_v.at[0]], rows[0], gsems[0])
    pltpu.async_copy(yr_hbm.at[sidx_v.at[1]], rows[1], gsems[1])

    def _body(g, carry):
        for b in range(5):
            k = 5 * g + b
            bn = (b + 2) % 5
            pltpu.make_async_copy(yr_hbm.at[sidx_v.at[k]], rows[b],
                                  gsems[b]).wait()
            pltpu.async_copy(rows[b], shared.at[didx_v.at[k]], ssems[b],
                             add=True)

            @pl.when(k >= 3)
            def _():
                pltpu.make_async_copy(rows[bn], shared.at[didx_v.at[k - 3]],
                                      ssems[bn]).wait()

            @pl.when(k + 2 < PNCH)
            def _():
                pltpu.async_copy(yr_hbm.at[sidx_v.at[k + 2]], rows[bn],
                                 gsems[bn])
        return carry

    lax.fori_loop(0, PNCH // 5, _body, 0)
    for kk in (PNCH - 3, PNCH - 2, PNCH - 1):
        pltpu.make_async_copy(rows[kk % 5], shared.at[didx_v.at[kk]],
                              ssems[kk % 5]).wait()
    plsc.subcore_barrier()
    nj = RPT // CH
    for j in range(nj):
        r = pl.multiple_of(row0 + j * CH, 8)
        buf = rows[j % 2]
        if j >= 2:
            rp = pl.multiple_of(row0 + (j - 2) * CH, 8)
            pltpu.make_async_copy(buf, out_hbm.at[cid, pl.ds(rp, CH)],
                                  gsems[j % 2]).wait()
        pltpu.sync_copy(shared.at[pl.ds(r, CH)], buf)
        pltpu.async_copy(buf, out_hbm.at[cid, pl.ds(r, CH)], gsems[j % 2])
    for j in range(nj - 2, nj):
        r = pl.multiple_of(row0 + j * CH, 8)
        pltpu.make_async_copy(rows[j % 2], out_hbm.at[cid, pl.ds(r, CH)],
                              gsems[j % 2]).wait()


def _prop_call(srcp4, dstp3t, yr):
    zfull = jnp.zeros((CH, DH), jnp.float32)
    return pl.kernel(
        _prop_kernel,
        out_type=jax.ShapeDtypeStruct((NC, NP, DH), jnp.float32),
        mesh=_sc_mesh(),
        compiler_params=pltpu.CompilerParams(use_tc_tiling_on_sc=False),
        scratch_types=[
            pltpu.VMEM((PNCH, CH), jnp.int32),
            pltpu.VMEM((PNCH, CH), jnp.int32),
            pltpu.VMEM((CH, DH), jnp.float32),
            pltpu.VMEM((CH, DH), jnp.float32),
            pltpu.VMEM((CH, DH), jnp.float32),
            pltpu.VMEM((CH, DH), jnp.float32),
            pltpu.VMEM((CH, DH), jnp.float32),
            pltpu.VMEM_SHARED((NP, DH), jnp.float32),
            pltpu.SemaphoreType.DMA,
            pltpu.SemaphoreType.DMA,
            pltpu.SemaphoreType.DMA,
            pltpu.SemaphoreType.DMA,
            pltpu.SemaphoreType.DMA,
            pltpu.SemaphoreType.DMA,
            pltpu.SemaphoreType.DMA,
            pltpu.SemaphoreType.DMA,
            pltpu.SemaphoreType.DMA,
            pltpu.SemaphoreType.DMA,
        ],
    )(srcp4, dstp3t, yr, zfull)


def _prep_body(x_ref, degt_ref, y_ref, dinv_ref):
    deg = 1.0 + degt_ref[:, 0:1] + degt_ref[:, 1:2]
    dinv = lax.rsqrt(deg)
    dinv_ref[...] = dinv
    y_ref[...] = x_ref[...] * dinv


def _prep_call(x, degt):
    return pl.pallas_call(
        _prep_body,
        grid=(NB,),
        in_specs=[
            pl.BlockSpec((BLK, D), lambda i: (i, 0)),
            pl.BlockSpec((BLK, 2), lambda i: (i, 0)),
        ],
        out_specs=[
            pl.BlockSpec((BLK, D), lambda i: (i, 0)),
            pl.BlockSpec((BLK, 1), lambda i: (i, 0)),
        ],
        out_shape=[
            jax.ShapeDtypeStruct((N, D), jnp.float32),
            jax.ShapeDtypeStruct((N, 1), jnp.float32),
        ],
    )(x, degt)


def _finish_body(zp_ref, y_ref, dinv_ref, x_ref, wexp_ref, bexp_ref,
                 w1_ref, b1_ref, w2_ref, b2_ref, out_ref,
                 zsc, g_sc, c_sc, wmp_sc, beta_sc):
    p = pl.program_id(0)
    i = pl.program_id(1)

    @pl.when(p == 0)
    def _phase0():
        by = jnp.concatenate([zp_ref[0], zp_ref[1]], axis=1)
        z = dinv_ref[...] * (by + y_ref[...])
        zsc[pl.ds(i * BLK, BLK), :] = z
        gpart = lax.dot_general(z, z, (((0,), (0,)), ((), ())),
                                preferred_element_type=jnp.float32)
        cpart = jnp.sum(z, axis=0, keepdims=True)

        @pl.when(i == 0)
        def _init():
            g_sc[...] = gpart
            c_sc[...] = cpart

        @pl.when(i > 0)
        def _acc():
            g_sc[...] += gpart
            c_sc[...] += cpart

    @pl.when((p == 1) & (i == 0))
    def _scalars():
        wall = wexp_ref[...]
        ball = bexp_ref[...]
        s = jnp.zeros((D, D), jnp.float32)
        t = jnp.zeros((1, D), jnp.float32)
        for e in range(NE):
            we = wall[e]
            s = s + lax.dot_general(we, we, (((1,), (1,)), ((), ())),
                                    preferred_element_type=jnp.float32)
            t = t + lax.dot_general(ball[0, e:e + 1], we,
                                    (((1,), (1,)), ((), ())),
                                    preferred_element_type=jnp.float32)
        c1 = c_sc[...]
        wrs = jnp.sum(jnp.sum(wall, axis=0), axis=1)[None]
        bsum = jnp.sum(ball)
        bsq = jnp.sum(ball * ball)
        M = float(NE * N * D)
        sum_all = BN * (jnp.sum(c1 * wrs) + N * bsum)
        sumsq_all = BN * BN * (jnp.sum(g_sc[...] * s)
                               + 2.0 * jnp.sum(c1 * t) + N * bsq)
        gm = sum_all / M
        gs = jnp.sqrt(jnp.maximum(sumsq_all - M * gm * gm, 0.0) / (M - 1.0))
        inv = 1.0 / (gs + 1e-8)
        wm = jnp.sum(wall, axis=0) * (1.0 / NE)
        bm = jnp.sum(ball[0], axis=0, keepdims=True) * (1.0 / NE)
        wmp_sc[...] = (BN * inv) * wm
        beta_sc[...] = (BN * bm - gm) * inv

    @pl.when(p == 1)
    def _phase1():
        z = zsc[pl.ds(i * BLK, BLK), :]
        mm = lax.dot_general(z, wmp_sc[...], (((1,), (0,)), ((), ())),
                             preferred_element_type=jnp.float32)
        h = (x_ref[...] + mm + beta_sc[...]) * BN
        a1 = lax.dot_general(h, w1_ref[...], (((1,), (0,)), ((), ())),
                             preferred_element_type=jnp.float32)
        a1 = jnp.maximum(a1 + b1_ref[...], 0.0)
        ff = lax.dot_general(a1, w2_ref[...], (((1,), (0,)), ((), ())),
                             preferred_element_type=jnp.float32) + b2_ref[...]
        out_ref[...] = (h + ff) * BN


def _finish_call(zp, y, dinv, x, W_exp, b_exp, W1, b1, W2, b2):
    b_exp3 = b_exp[None]
    return pl.pallas_call(
        _finish_body,
        grid=(2, NB),
        in_specs=[
            pl.BlockSpec((NC, BLK, DH), lambda p, i: (0, i, 0)),
            pl.BlockSpec((BLK, D), lambda p, i: (i, 0)),
            pl.BlockSpec((BLK, 1), lambda p, i: (i, 0)),
            pl.BlockSpec((BLK, D), lambda p, i: (i, 0)),
            pl.BlockSpec((NE, D, D), lambda p, i: (0, 0, 0)),
            pl.BlockSpec((1, NE, D), lambda p, i: (0, 0, 0)),
            pl.BlockSpec((D, 2 * D), lambda p, i: (0, 0)),
            pl.BlockSpec((1, 2 * D), lambda p, i: (0, 0)),
            pl.BlockSpec((2 * D, D), lambda p, i: (0, 0)),
            pl.BlockSpec((1, D), lambda p, i: (0, 0)),
        ],
        out_specs=pl.BlockSpec((BLK, D), lambda p, i: (i, 0)),
        out_shape=jax.ShapeDtypeStruct((N, D), jnp.float32),
        scratch_shapes=[
            pltpu.VMEM((N, D), jnp.float32),
            pltpu.VMEM((D, D), jnp.float32),
            pltpu.VMEM((1, D), jnp.float32),
            pltpu.VMEM((D, D), jnp.float32),
            pltpu.VMEM((1, D), jnp.float32),
        ],
    )(zp[:, :N], y, dinv, x, W_exp, b_exp3, W1, b1[None], W2, b2[None])


def kernel(x, edge_index, W_exp, b_exp, W1, b1, W2, b2):
    src = edge_index[0]
    dst = edge_index[1]
    pad = EP - E
    srcp = jnp.concatenate([src, jnp.zeros((pad,), jnp.int32)])
    dstp = jnp.concatenate(
        [dst, N + (jnp.arange(pad, dtype=jnp.int32) % (NP - N))])

    dstp3 = dstp.reshape(NW, DNCH, CH)
    srcp_t = srcp.reshape(NS, PNCH, CH)
    srcp4 = jnp.stack([2 * srcp_t, 2 * srcp_t + 1])
    dstp3t = dstp.reshape(NS, PNCH, CH)

    degp = _deg_call(dstp3)
    degt = jnp.stack([degp[0, :N], degp[1, :N]], axis=1)
    y, dinv = _prep_call(x, degt)

    yr = y.reshape(2 * N, DH)
    zp = _prop_call(srcp4, dstp3t, yr)
    return _finish_call(zp, y, dinv, x, W_exp, b_exp, W1, b1, W2, b2)


# speedup vs baseline: 1.4189x; 1.3977x over previous
import functools

import jax
import jax.numpy as jnp
from jax import lax
from jax.experimental import pallas as pl
from jax.experimental.pallas import tpu as pltpu
from jax.experimental.pallas import tpu_sc as plsc

N = 10000
D = 128
E = 160000
NE = 8
BN = float(1.0 / (1.0 + 1e-5) ** 0.5)
NC = 2
NS = 16
NW = NC * NS
NP = 10240
EP = 163840
CH = 128
DNCH = EP // NW // CH
PNCH = EP // NS // CH
RPT = NP // NS
DH = D // NC

BLK = 1000
NB = N // BLK


def _sc_mesh():
    return plsc.VectorSubcoreMesh(core_axis_name="c", subcore_axis_name="s")


def _deg_kernel(dstp_hbm, zrow_hbm, out_hbm, idx_v, ones_v, shared,
                sem0, sem1):
    cid = lax.axis_index("c")
    sid = lax.axis_index("s")
    wid = sid * NC + cid

    def _ones(i, carry):
        ones_v[pl.ds(i * 16, 16)] = jnp.ones((16,), jnp.float32)
        return carry

    lax.fori_loop(0, CH // 16, _ones, 0)

    row0 = pl.multiple_of(sid * RPT, 8)
    pltpu.sync_copy(zrow_hbm, shared.at[pl.ds(row0, RPT)])
    pltpu.sync_copy(dstp_hbm.at[wid], idx_v)
    plsc.subcore_barrier()

    sems = [sem0, sem1]
    pltpu.async_copy(ones_v, shared.at[idx_v.at[0]], sem0, add=True)
    pltpu.async_copy(ones_v, shared.at[idx_v.at[1]], sem1, add=True)

    def _body(g, carry):
        for b in range(2):
            k = 2 * g + b
            pltpu.make_async_copy(ones_v, shared.at[idx_v.at[k]],
                                  sems[b]).wait()

            @pl.when(k + 2 < DNCH)
            def _():
                pltpu.async_copy(ones_v, shared.at[idx_v.at[k + 2]], sems[b],
                                 add=True)
        return carry

    lax.fori_loop(0, DNCH // 2, _body, 0)
    plsc.subcore_barrier()
    pltpu.sync_copy(shared.at[pl.ds(row0, RPT)],
                    out_hbm.at[cid, pl.ds(row0, RPT)])


def _deg_call(dstp3):
    zrow = jnp.zeros((RPT,), jnp.float32)
    return pl.kernel(
        _deg_kernel,
        out_type=jax.ShapeDtypeStruct((NC, NP), jnp.float32),
        mesh=_sc_mesh(),
        scratch_types=[
            pltpu.VMEM((DNCH, CH), jnp.int32),
            pltpu.VMEM((CH,), jnp.float32),
            pltpu.VMEM_SHARED((NP,), jnp.float32),
            pltpu.SemaphoreType.DMA,
            pltpu.SemaphoreType.DMA,
        ],
    )(dstp3, zrow)


def _prop_kernel(srcp_hbm, dstp_hbm, yr_hbm, zfull_hbm, out_hbm,
                 sidx_v, didx_v, rows0_v, rows1_v, rows2_v, rows3_v, rows4_v,
                 shared, gsem0, gsem1, gsem2, gsem3, gsem4,
                 ssem0, ssem1, ssem2, ssem3, ssem4):
    cid = lax.axis_index("c")
    sid = lax.axis_index("s")

    row0 = pl.multiple_of(sid * RPT, 8)
    pltpu.sync_copy(zfull_hbm, rows0_v)
    for j in range(RPT // CH):
        pltpu.sync_copy(rows0_v,
                        shared.at[pl.ds(pl.multiple_of(row0 + j * CH, 8), CH)])
    pltpu.sync_copy(srcp_hbm.at[cid, sid], sidx_v)
    pltpu.sync_copy(dstp_hbm.at[sid], didx_v)
    plsc.subcore_barrier()

    rows = [rows0_v, rows1_v, rows2_v, rows3_v, rows4_v]
    gsems = [gsem0, gsem1, gsem2, gsem3, gsem4]
    ssems = [ssem0, ssem1, ssem2, ssem3, ssem4]

    pltpu.async_copy(yr_hbm.at[sidx_v.at[0]], rows[0], gsems[0])
    pltpu.async_copy(yr_hbm.at[sidx_v.at[1]], rows[1], gsems[1])

    def _body(g, carry):
        for b in range(5):
            k = 5 * g + b
            bn = (b + 2) % 5
            pltpu.make_async_copy(yr_hbm.at[sidx_v.at[k]], rows[b],
                                  gsems[b]).wait()
            pltpu.async_copy(rows[b], shared.at[didx_v.at[k]], ssems[b],
                             add=True)

            @pl.when(k >= 3)
            def _():
                pltpu.make_async_copy(rows[bn], shared.at[didx_v.at[k - 3]],
                                      ssems[bn]).wait()

            @pl.when(k + 2 < PNCH)
            def _():
                pltpu.async_copy(yr_hbm.at[sidx_v.at[k + 2]], rows[bn],
                                 gsems[bn])
        return carry

    lax.fori_loop(0, PNCH // 5, _body, 0)
    for kk in (PNCH - 3, PNCH - 2, PNCH - 1):
        pltpu.make_async_copy(rows[kk % 5], shared.at[didx_v.at[kk]],
                              ssems[kk % 5]).wait()
    plsc.subcore_barrier()
    nj = RPT // CH
    for j in range(nj):
        r = pl.multiple_of(row0 + j * CH, 8)
        buf = rows[j % 2]
        if j >= 2:
            rp = pl.multiple_of(row0 + (j - 2) * CH, 8)
            pltpu.make_async_copy(buf, out_hbm.at[cid, pl.ds(rp, CH)],
                                  gsems[j % 2]).wait()
        pltpu.sync_copy(shared.at[pl.ds(r, CH)], buf)
        pltpu.async_copy(buf, out_hbm.at[cid, pl.ds(r, CH)], gsems[j % 2])
    for j in range(nj - 2, nj):
        r = pl.multiple_of(row0 + j * CH, 8)
        pltpu.make_async_copy(rows[j % 2], out_hbm.at[cid, pl.ds(r, CH)],
                              gsems[j % 2]).wait()


def _prop_call(srcp4, dstp3t, yr):
    zfull = jnp.zeros((CH, DH), jnp.bfloat16)
    return pl.kernel(
        _prop_kernel,
        out_type=jax.ShapeDtypeStruct((NC, NP, DH), jnp.bfloat16),
        mesh=_sc_mesh(),
        compiler_params=pltpu.CompilerParams(use_tc_tiling_on_sc=False),
        scratch_types=[
            pltpu.VMEM((PNCH, CH), jnp.int32),
            pltpu.VMEM((PNCH, CH), jnp.int32),
            pltpu.VMEM((CH, DH), jnp.bfloat16),
            pltpu.VMEM((CH, DH), jnp.bfloat16),
            pltpu.VMEM((CH, DH), jnp.bfloat16),
            pltpu.VMEM((CH, DH), jnp.bfloat16),
            pltpu.VMEM((CH, DH), jnp.bfloat16),
            pltpu.VMEM_SHARED((NP, DH), jnp.bfloat16),
            pltpu.SemaphoreType.DMA,
            pltpu.SemaphoreType.DMA,
            pltpu.SemaphoreType.DMA,
            pltpu.SemaphoreType.DMA,
            pltpu.SemaphoreType.DMA,
            pltpu.SemaphoreType.DMA,
            pltpu.SemaphoreType.DMA,
            pltpu.SemaphoreType.DMA,
            pltpu.SemaphoreType.DMA,
            pltpu.SemaphoreType.DMA,
        ],
    )(srcp4, dstp3t, yr, zfull)


def _prep_body(x_ref, degt_ref, yb_ref, dinv_ref):
    deg = 1.0 + degt_ref[:, 0:1] + degt_ref[:, 1:2]
    dinv = lax.rsqrt(deg)
    dinv_ref[...] = dinv
    yb_ref[...] = (x_ref[...] * dinv).astype(jnp.bfloat16)


def _prep_call(x, degt):
    return pl.pallas_call(
        _prep_body,
        grid=(NB,),
        in_specs=[
            pl.BlockSpec((BLK, D), lambda i: (i, 0)),
            pl.BlockSpec((BLK, 2), lambda i: (i, 0)),
        ],
        out_specs=[
            pl.BlockSpec((BLK, D), lambda i: (i, 0)),
            pl.BlockSpec((BLK, 1), lambda i: (i, 0)),
        ],
        out_shape=[
            jax.ShapeDtypeStruct((N, D), jnp.bfloat16),
            jax.ShapeDtypeStruct((N, 1), jnp.float32),
        ],
    )(x, degt)


def _finish_body(zp_ref, dinv_ref, x_ref, wexp_ref, bexp_ref,
                 w1_ref, b1_ref, w2_ref, b2_ref, out_ref,
                 zsc, g_sc, c_sc, wmp_sc, beta_sc):
    p = pl.program_id(0)
    i = pl.program_id(1)

    @pl.when(p == 0)
    def _phase0():
        by = jnp.concatenate([zp_ref[0], zp_ref[1]],
                             axis=1).astype(jnp.float32)
        dinv = dinv_ref[...]
        z = dinv * by + (dinv * dinv) * x_ref[...]
        zsc[pl.ds(i * BLK, BLK), :] = z
        gpart = lax.dot_general(z, z, (((0,), (0,)), ((), ())),
                                preferred_element_type=jnp.float32)
        cpart = jnp.sum(z, axis=0, keepdims=True)

        @pl.when(i == 0)
        def _init():
            g_sc[...] = gpart
            c_sc[...] = cpart

        @pl.when(i > 0)
        def _acc():
            g_sc[...] += gpart
            c_sc[...] += cpart

    @pl.when((p == 1) & (i == 0))
    def _scalars():
        wall = wexp_ref[...]
        ball = bexp_ref[...]
        s = jnp.zeros((D, D), jnp.float32)
        t = jnp.zeros((1, D), jnp.float32)
        for e in range(NE):
            we = wall[e]
            s = s + lax.dot_general(we, we, (((1,), (1,)), ((), ())),
                                    preferred_element_type=jnp.float32)
            t = t + lax.dot_general(ball[0, e:e + 1], we,
                                    (((1,), (1,)), ((), ())),
                                    preferred_element_type=jnp.float32)
        c1 = c_sc[...]
        wrs = jnp.sum(jnp.sum(wall, axis=0), axis=1)[None]
        bsum = jnp.sum(ball)
        bsq = jnp.sum(ball * ball)
        M = float(NE * N * D)
        sum_all = BN * (jnp.sum(c1 * wrs) + N * bsum)
        sumsq_all = BN * BN * (jnp.sum(g_sc[...] * s)
                               + 2.0 * jnp.sum(c1 * t) + N * bsq)
        gm = sum_all / M
        gs = jnp.sqrt(jnp.maximum(sumsq_all - M * gm * gm, 0.0) / (M - 1.0))
        inv = 1.0 / (gs + 1e-8)
        wm = jnp.sum(wall, axis=0) * (1.0 / NE)
        bm = jnp.sum(ball[0], axis=0, keepdims=True) * (1.0 / NE)
        wmp_sc[...] = (BN * inv) * wm
        beta_sc[...] = (BN * bm - gm) * inv

    @pl.when(p == 1)
    def _phase1():
        z = zsc[pl.ds(i * BLK, BLK), :]
        mm = lax.dot_general(z, wmp_sc[...], (((1,), (0,)), ((), ())),
                             preferred_element_type=jnp.float32)
        h = (x_ref[...] + mm + beta_sc[...]) * BN
        a1 = lax.dot_general(h, w1_ref[...], (((1,), (0,)), ((), ())),
                             preferred_element_type=jnp.float32)
        a1 = jnp.maximum(a1 + b1_ref[...], 0.0)
        ff = lax.dot_general(a1, w2_ref[...], (((1,), (0,)), ((), ())),
                             preferred_element_type=jnp.float32) + b2_ref[...]
        out_ref[...] = (h + ff) * BN


def _finish_call(zp, dinv, x, W_exp, b_exp, W1, b1, W2, b2):
    b_exp3 = b_exp[None]
    return pl.pallas_call(
        _finish_body,
        grid=(2, NB),
        in_specs=[
            pl.BlockSpec((NC, BLK, DH), lambda p, i: (0, i, 0)),
            pl.BlockSpec((BLK, 1), lambda p, i: (i, 0)),
            pl.BlockSpec((BLK, D), lambda p, i: (i, 0)),
            pl.BlockSpec((NE, D, D), lambda p, i: (0, 0, 0)),
            pl.BlockSpec((1, NE, D), lambda p, i: (0, 0, 0)),
            pl.BlockSpec((D, 2 * D), lambda p, i: (0, 0)),
            pl.BlockSpec((1, 2 * D), lambda p, i: (0, 0)),
            pl.BlockSpec((2 * D, D), lambda p, i: (0, 0)),
            pl.BlockSpec((1, D), lambda p, i: (0, 0)),
        ],
        out_specs=pl.BlockSpec((BLK, D), lambda p, i: (i, 0)),
        out_shape=jax.ShapeDtypeStruct((N, D), jnp.float32),
        scratch_shapes=[
            pltpu.VMEM((N, D), jnp.float32),
            pltpu.VMEM((D, D), jnp.float32),
            pltpu.VMEM((1, D), jnp.float32),
            pltpu.VMEM((D, D), jnp.float32),
            pltpu.VMEM((1, D), jnp.float32),
        ],
    )(zp[:, :N], dinv, x, W_exp, b_exp3, W1, b1[None], W2, b2[None])


def kernel(x, edge_index, W_exp, b_exp, W1, b1, W2, b2):
    src = edge_index[0]
    dst = edge_index[1]
    pad = EP - E
    srcp = jnp.concatenate([src, jnp.zeros((pad,), jnp.int32)])
    dstp = jnp.concatenate(
        [dst, N + (jnp.arange(pad, dtype=jnp.int32) % (NP - N))])

    dstp3 = dstp.reshape(NW, DNCH, CH)
    srcp_t = srcp.reshape(NS, PNCH, CH)
    srcp4 = jnp.stack([2 * srcp_t, 2 * srcp_t + 1])
    dstp3t = dstp.reshape(NS, PNCH, CH)

    degp = _deg_call(dstp3)
    degt = jnp.stack([degp[0, :N], degp[1, :N]], axis=1)
    yb, dinv = _prep_call(x, degt)

    yr = yb.reshape(2 * N, DH)
    zp = _prop_call(srcp4, dstp3t, yr)
    return _finish_call(zp, dinv, x, W_exp, b_exp, W1, b1, W2, b2)


# speedup vs baseline: 1.5206x; 1.0716x over previous
import functools

import jax
import jax.numpy as jnp
from jax import lax
from jax.experimental import pallas as pl
from jax.experimental.pallas import tpu as pltpu
from jax.experimental.pallas import tpu_sc as plsc

N = 10000
D = 128
E = 160000
NE = 8
BN = float(1.0 / (1.0 + 1e-5) ** 0.5)
NC = 2
NS = 16
NW = NC * NS
NP = 10240
EP = 163840
CH = 128
DNCH = EP // NW // CH
PNCH = EP // NS // CH
RPT = NP // NS
DH = D // NC

BLK = 2000
NB = N // BLK


def _sc_mesh():
    return plsc.VectorSubcoreMesh(core_axis_name="c", subcore_axis_name="s")


def _deg_kernel(dstp_hbm, zrow_hbm, out_hbm, idx_v, ones_v, shared,
                sem0, sem1):
    cid = lax.axis_index("c")
    sid = lax.axis_index("s")
    wid = sid * NC + cid

    def _ones(i, carry):
        ones_v[pl.ds(i * 16, 16)] = jnp.ones((16,), jnp.float32)
        return carry

    lax.fori_loop(0, CH // 16, _ones, 0)

    row0 = pl.multiple_of(sid * RPT, 8)
    pltpu.sync_copy(zrow_hbm, shared.at[pl.ds(row0, RPT)])
    pltpu.sync_copy(dstp_hbm.at[wid], idx_v)
    plsc.subcore_barrier()

    sems = [sem0, sem1]
    pltpu.async_copy(ones_v, shared.at[idx_v.at[0]], sem0, add=True)
    pltpu.async_copy(ones_v, shared.at[idx_v.at[1]], sem1, add=True)

    def _body(g, carry):
        for b in range(2):
            k = 2 * g + b
            pltpu.make_async_copy(ones_v, shared.at[idx_v.at[k]],
                                  sems[b]).wait()

            @pl.when(k + 2 < DNCH)
            def _():
                pltpu.async_copy(ones_v, shared.at[idx_v.at[k + 2]], sems[b],
                                 add=True)
        return carry

    lax.fori_loop(0, DNCH // 2, _body, 0)
    plsc.subcore_barrier()
    pltpu.sync_copy(shared.at[pl.ds(row0, RPT)],
                    out_hbm.at[cid, pl.ds(row0, RPT)])


def _deg_call(dstp3):
    zrow = jnp.zeros((RPT,), jnp.float32)
    return pl.kernel(
        _deg_kernel,
        out_type=jax.ShapeDtypeStruct((NC, NP), jnp.float32),
        mesh=_sc_mesh(),
        scratch_types=[
            pltpu.VMEM((DNCH, CH), jnp.int32),
            pltpu.VMEM((CH,), jnp.float32),
            pltpu.VMEM_SHARED((NP,), jnp.float32),
            pltpu.SemaphoreType.DMA,
            pltpu.SemaphoreType.DMA,
        ],
    )(dstp3, zrow)


def _prop_kernel(srcp_hbm, dstp_hbm, yr_hbm, zfull_hbm, out_hbm,
                 sidx_v, didx_v, rows0_v, rows1_v, rows2_v, rows3_v, rows4_v,
                 shared, gsem0, gsem1, gsem2, gsem3, gsem4,
                 ssem0, ssem1, ssem2, ssem3, ssem4):
    cid = lax.axis_index("c")
    sid = lax.axis_index("s")

    row0 = pl.multiple_of(sid * RPT, 8)
    pltpu.sync_copy(zfull_hbm, rows0_v)
    for j in range(RPT // CH):
        pltpu.sync_copy(rows0_v,
                        shared.at[pl.ds(pl.multiple_of(row0 + j * CH, 8), CH)])
    pltpu.sync_copy(srcp_hbm.at[cid, sid], sidx_v)
    pltpu.sync_copy(dstp_hbm.at[sid], didx_v)
    plsc.subcore_barrier()

    rows = [rows0_v, rows1_v, rows2_v, rows3_v, rows4_v]
    gsems = [gsem0, gsem1, gsem2, gsem3, gsem4]
    ssems = [ssem0, ssem1, ssem2, ssem3, ssem4]

    pltpu.async_copy(yr_hbm.at[sidx_v.at[0]], rows[0], gsems[0])
    pltpu.async_copy(yr_hbm.at[sidx_v.at[1]], rows[1], gsems[1])

    def _body(g, carry):
        for b in range(5):
            k = 5 * g + b
            bn = (b + 2) % 5
            pltpu.make_async_copy(yr_hbm.at[sidx_v.at[k]], rows[b],
                                  gsems[b]).wait()
            pltpu.async_copy(rows[b], shared.at[didx_v.at[k]], ssems[b],
                             add=True)

            @pl.when(k >= 3)
            def _():
                pltpu.make_async_copy(rows[bn], shared.at[didx_v.at[k - 3]],
                                      ssems[bn]).wait()

            @pl.when(k + 2 < PNCH)
            def _():
                pltpu.async_copy(yr_hbm.at[sidx_v.at[k + 2]], rows[bn],
                                 gsems[bn])
        return carry

    lax.fori_loop(0, PNCH // 5, _body, 0)
    for kk in (PNCH - 3, PNCH - 2, PNCH - 1):
        pltpu.make_async_copy(rows[kk % 5], shared.at[didx_v.at[kk]],
                              ssems[kk % 5]).wait()
    plsc.subcore_barrier()
    nj = RPT // CH
    for j in range(nj):
        r = pl.multiple_of(row0 + j * CH, 8)
        buf = rows[j % 2]
        if j >= 2:
            rp = pl.multiple_of(row0 + (j - 2) * CH, 8)
            pltpu.make_async_copy(buf, out_hbm.at[cid, pl.ds(rp, CH)],
                                  gsems[j % 2]).wait()
        pltpu.sync_copy(shared.at[pl.ds(r, CH)], buf)
        pltpu.async_copy(buf, out_hbm.at[cid, pl.ds(r, CH)], gsems[j % 2])
    for j in range(nj - 2, nj):
        r = pl.multiple_of(row0 + j * CH, 8)
        pltpu.make_async_copy(rows[j % 2], out_hbm.at[cid, pl.ds(r, CH)],
                              gsems[j % 2]).wait()


def _prop_call(srcp4, dstp3t, yr):
    zfull = jnp.zeros((CH, DH), jnp.bfloat16)
    return pl.kernel(
        _prop_kernel,
        out_type=jax.ShapeDtypeStruct((NC, NP, DH), jnp.bfloat16),
        mesh=_sc_mesh(),
        compiler_params=pltpu.CompilerParams(use_tc_tiling_on_sc=False),
        scratch_types=[
            pltpu.VMEM((PNCH, CH), jnp.int32),
            pltpu.VMEM((PNCH, CH), jnp.int32),
            pltpu.VMEM((CH, DH), jnp.bfloat16),
            pltpu.VMEM((CH, DH), jnp.bfloat16),
            pltpu.VMEM((CH, DH), jnp.bfloat16),
            pltpu.VMEM((CH, DH), jnp.bfloat16),
            pltpu.VMEM((CH, DH), jnp.bfloat16),
            pltpu.VMEM_SHARED((NP, DH), jnp.bfloat16),
            pltpu.SemaphoreType.DMA,
            pltpu.SemaphoreType.DMA,
            pltpu.SemaphoreType.DMA,
            pltpu.SemaphoreType.DMA,
            pltpu.SemaphoreType.DMA,
            pltpu.SemaphoreType.DMA,
            pltpu.SemaphoreType.DMA,
            pltpu.SemaphoreType.DMA,
            pltpu.SemaphoreType.DMA,
            pltpu.SemaphoreType.DMA,
        ],
    )(srcp4, dstp3t, yr, zfull)


def _prep_body(x_ref, degt_ref, yb_ref, dinv_ref):
    deg = 1.0 + degt_ref[:, 0:1] + degt_ref[:, 1:2]
    dinv = lax.rsqrt(deg)
    dinv_ref[...] = dinv
    yb_ref[...] = (x_ref[...] * dinv).astype(jnp.bfloat16)


def _prep_call(x, degt):
    return pl.pallas_call(
        _prep_body,
        grid=(NB,),
        in_specs=[
            pl.BlockSpec((BLK, D), lambda i: (i, 0)),
            pl.BlockSpec((BLK, 2), lambda i: (i, 0)),
        ],
        out_specs=[
            pl.BlockSpec((BLK, D), lambda i: (i, 0)),
            pl.BlockSpec((BLK, 1), lambda i: (i, 0)),
        ],
        out_shape=[
            jax.ShapeDtypeStruct((N, D), jnp.bfloat16),
            jax.ShapeDtypeStruct((N, 1), jnp.float32),
        ],
    )(x, degt)


def _finish_body(zp_ref, dinv_ref, x_ref, wexp_ref, bexp_ref,
                 w1_ref, b1_ref, w2_ref, b2_ref, out_ref,
                 zsc, g_sc, c_sc, wmp_sc, beta_sc):
    p = pl.program_id(0)
    i = pl.program_id(1)

    @pl.when(p == 0)
    def _phase0():
        by = jnp.concatenate([zp_ref[0], zp_ref[1]],
                             axis=1).astype(jnp.float32)
        dinv = dinv_ref[...]
        z = dinv * by + (dinv * dinv) * x_ref[...]
        zsc[pl.ds(i * BLK, BLK), :] = z
        gpart = lax.dot_general(z, z, (((0,), (0,)), ((), ())),
                                preferred_element_type=jnp.float32)
        cpart = jnp.sum(z, axis=0, keepdims=True)

        @pl.when(i == 0)
        def _init():
            g_sc[...] = gpart
            c_sc[...] = cpart

        @pl.when(i > 0)
        def _acc():
            g_sc[...] += gpart
            c_sc[...] += cpart

    @pl.when((p == 1) & (i == 0))
    def _scalars():
        wall = wexp_ref[...]
        ball = bexp_ref[...]
        s = jnp.zeros((D, D), jnp.float32)
        t = jnp.zeros((1, D), jnp.float32)
        for e in range(NE):
            we = wall[e]
            s = s + lax.dot_general(we, we, (((1,), (1,)), ((), ())),
                                    preferred_element_type=jnp.float32)
            t = t + lax.dot_general(ball[0, e:e + 1], we,
                                    (((1,), (1,)), ((), ())),
                                    preferred_element_type=jnp.float32)
        c1 = c_sc[...]
        wrs = jnp.sum(jnp.sum(wall, axis=0), axis=1)[None]
        bsum = jnp.sum(ball)
        bsq = jnp.sum(ball * ball)
        M = float(NE * N * D)
        sum_all = BN * (jnp.sum(c1 * wrs) + N * bsum)
        sumsq_all = BN * BN * (jnp.sum(g_sc[...] * s)
                               + 2.0 * jnp.sum(c1 * t) + N * bsq)
        gm = sum_all / M
        gs = jnp.sqrt(jnp.maximum(sumsq_all - M * gm * gm, 0.0) / (M - 1.0))
        inv = 1.0 / (gs + 1e-8)
        wm = jnp.sum(wall, axis=0) * (1.0 / NE)
        bm = jnp.sum(ball[0], axis=0, keepdims=True) * (1.0 / NE)
        wmp_sc[...] = (BN * inv) * wm
        beta_sc[...] = (BN * bm - gm) * inv

    @pl.when(p == 1)
    def _phase1():
        z = zsc[pl.ds(i * BLK, BLK), :]
        mm = lax.dot_general(z, wmp_sc[...], (((1,), (0,)), ((), ())),
                             preferred_element_type=jnp.float32)
        h = (x_ref[...] + mm + beta_sc[...]) * BN
        a1 = lax.dot_general(h, w1_ref[...], (((1,), (0,)), ((), ())),
                             preferred_element_type=jnp.float32)
        a1 = jnp.maximum(a1 + b1_ref[...], 0.0)
        ff = lax.dot_general(a1, w2_ref[...], (((1,), (0,)), ((), ())),
                             preferred_element_type=jnp.float32) + b2_ref[...]
        out_ref[...] = (h + ff) * BN


def _finish_call(zp, dinv, x, W_exp, b_exp, W1, b1, W2, b2):
    b_exp3 = b_exp[None]
    return pl.pallas_call(
        _finish_body,
        grid=(2, NB),
        in_specs=[
            pl.BlockSpec((NC, BLK, DH), lambda p, i: (0, i, 0)),
            pl.BlockSpec((BLK, 1), lambda p, i: (i, 0)),
            pl.BlockSpec((BLK, D), lambda p, i: (i, 0)),
            pl.BlockSpec((NE, D, D), lambda p, i: (0, 0, 0)),
            pl.BlockSpec((1, NE, D), lambda p, i: (0, 0, 0)),
            pl.BlockSpec((D, 2 * D), lambda p, i: (0, 0)),
            pl.BlockSpec((1, 2 * D), lambda p, i: (0, 0)),
            pl.BlockSpec((2 * D, D), lambda p, i: (0, 0)),
            pl.BlockSpec((1, D), lambda p, i: (0, 0)),
        ],
        out_specs=pl.BlockSpec((BLK, D), lambda p, i: (i, 0)),
        out_shape=jax.ShapeDtypeStruct((N, D), jnp.float32),
        scratch_shapes=[
            pltpu.VMEM((N, D), jnp.float32),
            pltpu.VMEM((D, D), jnp.float32),
            pltpu.VMEM((1, D), jnp.float32),
            pltpu.VMEM((D, D), jnp.float32),
            pltpu.VMEM((1, D), jnp.float32),
        ],
    )(zp, dinv, x, W_exp, b_exp3, W1, b1[None], W2, b2[None])


def kernel(x, edge_index, W_exp, b_exp, W1, b1, W2, b2):
    src = edge_index[0]
    dst = edge_index[1]
    pad = EP - E
    srcp = jnp.concatenate([src, jnp.zeros((pad,), jnp.int32)])
    dstp = jnp.concatenate(
        [dst, N + (jnp.arange(pad, dtype=jnp.int32) % (NP - N))])

    dstp3 = dstp.reshape(NW, DNCH, CH)
    srcp_t = srcp.reshape(NS, PNCH, CH)
    srcp4 = jnp.stack([2 * srcp_t, 2 * srcp_t + 1])
    dstp3t = dstp.reshape(NS, PNCH, CH)

    degp = _deg_call(dstp3)
    degt = jnp.stack([degp[0, :N], degp[1, :N]], axis=1)
    yb, dinv = _prep_call(x, degt)

    yr = yb.reshape(2 * N, DH)
    zp = _prop_call(srcp4, dstp3t, yr)
    return _finish_call(zp, dinv, x, W_exp, b_exp, W1, b1, W2, b2)


# speedup vs baseline: 1.5633x; 1.0281x over previous
import functools

import jax
import jax.numpy as jnp
from jax import lax
from jax.experimental import pallas as pl
from jax.experimental.pallas import tpu as pltpu
from jax.experimental.pallas import tpu_sc as plsc

N = 10000
D = 128
E = 160000
NE = 8
BN = float(1.0 / (1.0 + 1e-5) ** 0.5)
NC = 2
NS = 16
NW = NC * NS
NP = 10240
EP = 163840
CH = 128
DNCH = EP // NW // CH
PNCH = EP // NS // CH
RPT = NP // NS
DH = D // NC

BLK = 2000
NB = N // BLK


def _sc_mesh():
    return plsc.VectorSubcoreMesh(core_axis_name="c", subcore_axis_name="s")


def _deg_kernel(dstp_hbm, zrow_hbm, out_hbm, idx_v, ones_v, shared,
                sem0, sem1):
    cid = lax.axis_index("c")
    sid = lax.axis_index("s")
    wid = sid * NC + cid

    def _ones(i, carry):
        ones_v[pl.ds(i * 16, 16)] = jnp.ones((16,), jnp.float32)
        return carry

    lax.fori_loop(0, CH // 16, _ones, 0)

    row0 = pl.multiple_of(sid * RPT, 8)
    pltpu.sync_copy(zrow_hbm, shared.at[pl.ds(row0, RPT)])
    pltpu.sync_copy(dstp_hbm.at[wid], idx_v)
    plsc.subcore_barrier()

    sems = [sem0, sem1]
    pltpu.async_copy(ones_v, shared.at[idx_v.at[0]], sem0, add=True)
    pltpu.async_copy(ones_v, shared.at[idx_v.at[1]], sem1, add=True)

    def _body(g, carry):
        for b in range(2):
            k = 2 * g + b
            pltpu.make_async_copy(ones_v, shared.at[idx_v.at[k]],
                                  sems[b]).wait()

            @pl.when(k + 2 < DNCH)
            def _():
                pltpu.async_copy(ones_v, shared.at[idx_v.at[k + 2]], sems[b],
                                 add=True)
        return carry

    lax.fori_loop(0, DNCH // 2, _body, 0)
    plsc.subcore_barrier()
    pltpu.sync_copy(shared.at[pl.ds(row0, RPT)],
                    out_hbm.at[cid, pl.ds(row0, RPT)])


def _deg_call(dstp3):
    zrow = jnp.zeros((RPT,), jnp.float32)
    return pl.kernel(
        _deg_kernel,
        out_type=jax.ShapeDtypeStruct((NC, NP), jnp.float32),
        mesh=_sc_mesh(),
        scratch_types=[
            pltpu.VMEM((DNCH, CH), jnp.int32),
            pltpu.VMEM((CH,), jnp.float32),
            pltpu.VMEM_SHARED((NP,), jnp.float32),
            pltpu.SemaphoreType.DMA,
            pltpu.SemaphoreType.DMA,
        ],
    )(dstp3, zrow)


def _prop_kernel(srcp_hbm, dstp_hbm, yr_hbm, zfull_hbm, out_hbm,
                 sidx_v, didx_v, rows0_v, rows1_v, rows2_v, rows3_v, rows4_v,
                 shared, gsem0, gsem1, gsem2, gsem3, gsem4,
                 ssem0, ssem1, ssem2, ssem3, ssem4):
    cid = lax.axis_index("c")
    sid = lax.axis_index("s")

    row0 = pl.multiple_of(sid * RPT, 8)
    pltpu.sync_copy(zfull_hbm, rows0_v)
    for j in range(RPT // CH):
        pltpu.sync_copy(rows0_v,
                        shared.at[pl.ds(pl.multiple_of(row0 + j * CH, 8), CH)])
    pltpu.sync_copy(srcp_hbm.at[cid, sid], sidx_v)
    pltpu.sync_copy(dstp_hbm.at[sid], didx_v)
    plsc.subcore_barrier()

    rows = [rows0_v, rows1_v, rows2_v, rows3_v, rows4_v]
    gsems = [gsem0, gsem1, gsem2, gsem3, gsem4]
    ssems = [ssem0, ssem1, ssem2, ssem3, ssem4]

    pltpu.async_copy(yr_hbm.at[sidx_v.at[0]], rows[0], gsems[0])
    pltpu.async_copy(yr_hbm.at[sidx_v.at[1]], rows[1], gsems[1])
    pltpu.async_copy(yr_hbm.at[sidx_v.at[2]], rows[2], gsems[2])

    def _body(g, carry):
        for b in range(5):
            k = 5 * g + b
            bn = (b + 3) % 5
            pltpu.make_async_copy(yr_hbm.at[sidx_v.at[k]], rows[b],
                                  gsems[b]).wait()
            pltpu.async_copy(rows[b], shared.at[didx_v.at[k]], ssems[b],
                             add=True)

            @pl.when(k >= 2)
            def _():
                pltpu.make_async_copy(rows[bn], shared.at[didx_v.at[k - 2]],
                                      ssems[bn]).wait()

            @pl.when(k + 3 < PNCH)
            def _():
                pltpu.async_copy(yr_hbm.at[sidx_v.at[k + 3]], rows[bn],
                                 gsems[bn])
        return carry

    lax.fori_loop(0, PNCH // 5, _body, 0)
    for kk in (PNCH - 2, PNCH - 1):
        pltpu.make_async_copy(rows[kk % 5], shared.at[didx_v.at[kk]],
                              ssems[kk % 5]).wait()
    plsc.subcore_barrier()
    nj = RPT // CH
    for j in range(nj):
        r = pl.multiple_of(row0 + j * CH, 8)
        buf = rows[j % 2]
        if j >= 2:
            rp = pl.multiple_of(row0 + (j - 2) * CH, 8)
            pltpu.make_async_copy(buf, out_hbm.at[cid, pl.ds(rp, CH)],
                                  gsems[j % 2]).wait()
        pltpu.sync_copy(shared.at[pl.ds(r, CH)], buf)
        pltpu.async_copy(buf, out_hbm.at[cid, pl.ds(r, CH)], gsems[j % 2])
    for j in range(nj - 2, nj):
        r = pl.multiple_of(row0 + j * CH, 8)
        pltpu.make_async_copy(rows[j % 2], out_hbm.at[cid, pl.ds(r, CH)],
                              gsems[j % 2]).wait()


def _prop_call(srcp4, dstp3t, yr):
    zfull = jnp.zeros((CH, DH), jnp.bfloat16)
    return pl.kernel(
        _prop_kernel,
        out_type=jax.ShapeDtypeStruct((NC, NP, DH), jnp.bfloat16),
        mesh=_sc_mesh(),
        compiler_params=pltpu.CompilerParams(use_tc_tiling_on_sc=False),
        scratch_types=[
            pltpu.VMEM((PNCH, CH), jnp.int32),
            pltpu.VMEM((PNCH, CH), jnp.int32),
            pltpu.VMEM((CH, DH), jnp.bfloat16),
            pltpu.VMEM((CH, DH), jnp.bfloat16),
            pltpu.VMEM((CH, DH), jnp.bfloat16),
            pltpu.VMEM((CH, DH), jnp.bfloat16),
            pltpu.VMEM((CH, DH), jnp.bfloat16),
            pltpu.VMEM_SHARED((NP, DH), jnp.bfloat16),
            pltpu.SemaphoreType.DMA,
            pltpu.SemaphoreType.DMA,
            pltpu.SemaphoreType.DMA,
            pltpu.SemaphoreType.DMA,
            pltpu.SemaphoreType.DMA,
            pltpu.SemaphoreType.DMA,
            pltpu.SemaphoreType.DMA,
            pltpu.SemaphoreType.DMA,
            pltpu.SemaphoreType.DMA,
            pltpu.SemaphoreType.DMA,
        ],
    )(srcp4, dstp3t, yr, zfull)


def _prep_body(x_ref, degt_ref, yb_ref, dinv_ref):
    deg = 1.0 + degt_ref[:, 0:1] + degt_ref[:, 1:2]
    dinv = lax.rsqrt(deg)
    dinv_ref[...] = dinv
    yb_ref[...] = (x_ref[...] * dinv).astype(jnp.bfloat16)


def _prep_call(x, degt):
    return pl.pallas_call(
        _prep_body,
        grid=(NB,),
        in_specs=[
            pl.BlockSpec((BLK, D), lambda i: (i, 0)),
            pl.BlockSpec((BLK, 2), lambda i: (i, 0)),
        ],
        out_specs=[
            pl.BlockSpec((BLK, D), lambda i: (i, 0)),
            pl.BlockSpec((BLK, 1), lambda i: (i, 0)),
        ],
        out_shape=[
            jax.ShapeDtypeStruct((N, D), jnp.bfloat16),
            jax.ShapeDtypeStruct((N, 1), jnp.float32),
        ],
    )(x, degt)


def _finish_body(zp_ref, dinv_ref, x_ref, wexp_ref, bexp_ref,
                 w1_ref, b1_ref, w2_ref, b2_ref, out_ref,
                 zsc, g_sc, c_sc, wmp_sc, beta_sc):
    p = pl.program_id(0)
    i = pl.program_id(1)

    @pl.when(p == 0)
    def _phase0():
        by = jnp.concatenate([zp_ref[0], zp_ref[1]],
                             axis=1).astype(jnp.float32)
        dinv = dinv_ref[...]
        z = dinv * by + (dinv * dinv) * x_ref[...]
        zsc[pl.ds(i * BLK, BLK), :] = z
        gpart = lax.dot_general(z, z, (((0,), (0,)), ((), ())),
                                preferred_element_type=jnp.float32)
        cpart = jnp.sum(z, axis=0, keepdims=True)

        @pl.when(i == 0)
        def _init():
            g_sc[...] = gpart
            c_sc[...] = cpart

        @pl.when(i > 0)
        def _acc():
            g_sc[...] += gpart
            c_sc[...] += cpart

    @pl.when((p == 1) & (i == 0))
    def _scalars():
        wall = wexp_ref[...]
        ball = bexp_ref[...]
        s = jnp.zeros((D, D), jnp.float32)
        t = jnp.zeros((1, D), jnp.float32)
        for e in range(NE):
            we = wall[e]
            s = s + lax.dot_general(we, we, (((1,), (1,)), ((), ())),
                                    preferred_element_type=jnp.float32)
            t = t + lax.dot_general(ball[0, e:e + 1], we,
                                    (((1,), (1,)), ((), ())),
                                    preferred_element_type=jnp.float32)
        c1 = c_sc[...]
        wrs = jnp.sum(jnp.sum(wall, axis=0), axis=1)[None]
        bsum = jnp.sum(ball)
        bsq = jnp.sum(ball * ball)
        M = float(NE * N * D)
        sum_all = BN * (jnp.sum(c1 * wrs) + N * bsum)
        sumsq_all = BN * BN * (jnp.sum(g_sc[...] * s)
                               + 2.0 * jnp.sum(c1 * t) + N * bsq)
        gm = sum_all / M
        gs = jnp.sqrt(jnp.maximum(sumsq_all - M * gm * gm, 0.0) / (M - 1.0))
        inv = 1.0 / (gs + 1e-8)
        wm = jnp.sum(wall, axis=0) * (1.0 / NE)
        bm = jnp.sum(ball[0], axis=0, keepdims=True) * (1.0 / NE)
        wmp_sc[...] = (BN * inv) * wm
        beta_sc[...] = (BN * bm - gm) * inv

    @pl.when(p == 1)
    def _phase1():
        z = zsc[pl.ds(i * BLK, BLK), :]
        mm = lax.dot_general(z, wmp_sc[...], (((1,), (0,)), ((), ())),
                             preferred_element_type=jnp.float32)
        h = (x_ref[...] + mm + beta_sc[...]) * BN
        a1 = lax.dot_general(h, w1_ref[...], (((1,), (0,)), ((), ())),
                             preferred_element_type=jnp.float32)
        a1 = jnp.maximum(a1 + b1_ref[...], 0.0)
        ff = lax.dot_general(a1, w2_ref[...], (((1,), (0,)), ((), ())),
                             preferred_element_type=jnp.float32) + b2_ref[...]
        out_ref[...] = (h + ff) * BN


def _finish_call(zp, dinv, x, W_exp, b_exp, W1, b1, W2, b2):
    b_exp3 = b_exp[None]
    return pl.pallas_call(
        _finish_body,
        grid=(2, NB),
        in_specs=[
            pl.BlockSpec((NC, BLK, DH), lambda p, i: (0, i, 0)),
            pl.BlockSpec((BLK, 1), lambda p, i: (i, 0)),
            pl.BlockSpec((BLK, D), lambda p, i: (i, 0)),
            pl.BlockSpec((NE, D, D), lambda p, i: (0, 0, 0)),
            pl.BlockSpec((1, NE, D), lambda p, i: (0, 0, 0)),
            pl.BlockSpec((D, 2 * D), lambda p, i: (0, 0)),
            pl.BlockSpec((1, 2 * D), lambda p, i: (0, 0)),
            pl.BlockSpec((2 * D, D), lambda p, i: (0, 0)),
            pl.BlockSpec((1, D), lambda p, i: (0, 0)),
        ],
        out_specs=pl.BlockSpec((BLK, D), lambda p, i: (i, 0)),
        out_shape=jax.ShapeDtypeStruct((N, D), jnp.float32),
        scratch_shapes=[
            pltpu.VMEM((N, D), jnp.float32),
            pltpu.VMEM((D, D), jnp.float32),
            pltpu.VMEM((1, D), jnp.float32),
            pltpu.VMEM((D, D), jnp.float32),
            pltpu.VMEM((1, D), jnp.float32),
        ],
    )(zp, dinv, x, W_exp, b_exp3, W1, b1[None], W2, b2[None])


def kernel(x, edge_index, W_exp, b_exp, W1, b1, W2, b2):
    src = edge_index[0]
    dst = edge_index[1]
    pad = EP - E
    srcp = jnp.concatenate([src, jnp.zeros((pad,), jnp.int32)])
    dstp = jnp.concatenate(
        [dst, N + (jnp.arange(pad, dtype=jnp.int32) % (NP - N))])

    dstp3 = dstp.reshape(NW, DNCH, CH)
    srcp_t = srcp.reshape(NS, PNCH, CH)
    srcp4 = jnp.stack([2 * srcp_t, 2 * srcp_t + 1])
    dstp3t = dstp.reshape(NS, PNCH, CH)

    degp = _deg_call(dstp3)
    degt = jnp.stack([degp[0, :N], degp[1, :N]], axis=1)
    yb, dinv = _prep_call(x, degt)

    yr = yb.reshape(2 * N, DH)
    zp = _prop_call(srcp4, dstp3t, yr)
    return _finish_call(zp, dinv, x, W_exp, b_exp, W1, b1, W2, b2)


# speedup vs baseline: 2.2321x; 1.4278x over previous
import functools

import jax
import jax.numpy as jnp
from jax import lax
from jax.experimental import pallas as pl
from jax.experimental.pallas import tpu as pltpu
from jax.experimental.pallas import tpu_sc as plsc

N = 10000
D = 128
E = 160000
NE = 8
BN = float(1.0 / (1.0 + 1e-5) ** 0.5)
NC = 2
NS = 16
NW = NC * NS
NP = 10240
EP = 163840
CH = 128
DNCH = EP // NW // CH
PNCH = EP // NS // CH
RPT = NP // NS
DH = D // NC

BLK = 2000
NB = N // BLK


def _sc_mesh():
    return plsc.VectorSubcoreMesh(core_axis_name="c", subcore_axis_name="s")


def _deg_kernel(dstp_hbm, zrow_hbm, out_hbm, idx_v, ones_v, shared,
                sem0, sem1):
    cid = lax.axis_index("c")
    sid = lax.axis_index("s")
    wid = sid * NC + cid

    def _ones(i, carry):
        ones_v[pl.ds(i * 16, 16)] = jnp.ones((16,), jnp.float32)
        return carry

    lax.fori_loop(0, CH // 16, _ones, 0)

    row0 = pl.multiple_of(sid * RPT, 8)
    pltpu.sync_copy(zrow_hbm, shared.at[pl.ds(row0, RPT)])
    pltpu.sync_copy(dstp_hbm.at[wid], idx_v)
    plsc.subcore_barrier()

    sems = [sem0, sem1]
    pltpu.async_copy(ones_v, shared.at[idx_v.at[0]], sem0, add=True)
    pltpu.async_copy(ones_v, shared.at[idx_v.at[1]], sem1, add=True)

    def _body(g, carry):
        for b in range(2):
            k = 2 * g + b
            pltpu.make_async_copy(ones_v, shared.at[idx_v.at[k]],
                                  sems[b]).wait()

            @pl.when(k + 2 < DNCH)
            def _():
                pltpu.async_copy(ones_v, shared.at[idx_v.at[k + 2]], sems[b],
                                 add=True)
        return carry

    lax.fori_loop(0, DNCH // 2, _body, 0)
    plsc.subcore_barrier()
    pltpu.sync_copy(shared.at[pl.ds(row0, RPT)],
                    out_hbm.at[cid, pl.ds(row0, RPT)])


def _deg_call(dstp3):
    zrow = jnp.zeros((RPT,), jnp.float32)
    return pl.kernel(
        _deg_kernel,
        out_type=jax.ShapeDtypeStruct((NC, NP), jnp.float32),
        mesh=_sc_mesh(),
        scratch_types=[
            pltpu.VMEM((DNCH, CH), jnp.int32),
            pltpu.VMEM((CH,), jnp.float32),
            pltpu.VMEM_SHARED((NP,), jnp.float32),
            pltpu.SemaphoreType.DMA,
            pltpu.SemaphoreType.DMA,
        ],
    )(dstp3, zrow)


def _prop_kernel(srcp_hbm, dstp_hbm, yb2_hbm, zfull_hbm, out_hbm,
                 sidx_v, didx_v, rows0_v, rows1_v, rows2_v, rows3_v, rows4_v,
                 shared_y, shared, gsem0, gsem1, gsem2, gsem3, gsem4,
                 ssem0, ssem1, ssem2, ssem3, ssem4):
    cid = lax.axis_index("c")
    sid = lax.axis_index("s")

    row0 = pl.multiple_of(sid * RPT, 8)
    pltpu.sync_copy(zfull_hbm, rows0_v)
    for j in range(RPT // CH):
        pltpu.sync_copy(rows0_v,
                        shared.at[pl.ds(pl.multiple_of(row0 + j * CH, 8), CH)])
    for j in range(RPT // CH):
        r = pl.multiple_of(row0 + j * CH, 8)
        buf = [rows1_v, rows2_v][j % 2]
        pltpu.sync_copy(yb2_hbm.at[cid, pl.ds(r, CH)], buf)
        pltpu.sync_copy(buf, shared_y.at[pl.ds(r, CH)])
    pltpu.sync_copy(srcp_hbm.at[sid], sidx_v)
    pltpu.sync_copy(dstp_hbm.at[sid], didx_v)
    plsc.subcore_barrier()

    rows = [rows0_v, rows1_v, rows2_v, rows3_v, rows4_v]
    gsems = [gsem0, gsem1, gsem2, gsem3, gsem4]
    ssems = [ssem0, ssem1, ssem2, ssem3, ssem4]

    pltpu.async_copy(shared_y.at[sidx_v.at[0]], rows[0], gsems[0])
    pltpu.async_copy(shared_y.at[sidx_v.at[1]], rows[1], gsems[1])
    pltpu.async_copy(shared_y.at[sidx_v.at[2]], rows[2], gsems[2])

    def _body(g, carry):
        for b in range(5):
            k = 5 * g + b
            bn = (b + 3) % 5
            pltpu.make_async_copy(shared_y.at[sidx_v.at[k]], rows[b],
                                  gsems[b]).wait()
            pltpu.async_copy(rows[b], shared.at[didx_v.at[k]], ssems[b],
                             add=True)

            @pl.when(k >= 2)
            def _():
                pltpu.make_async_copy(rows[bn], shared.at[didx_v.at[k - 2]],
                                      ssems[bn]).wait()

            @pl.when(k + 3 < PNCH)
            def _():
                pltpu.async_copy(shared_y.at[sidx_v.at[k + 3]], rows[bn],
                                 gsems[bn])
        return carry

    lax.fori_loop(0, PNCH // 5, _body, 0)
    for kk in (PNCH - 2, PNCH - 1):
        pltpu.make_async_copy(rows[kk % 5], shared.at[didx_v.at[kk]],
                              ssems[kk % 5]).wait()
    plsc.subcore_barrier()
    nj = RPT // CH
    for j in range(nj):
        r = pl.multiple_of(row0 + j * CH, 8)
        buf = rows[j % 2]
        if j >= 2:
            rp = pl.multiple_of(row0 + (j - 2) * CH, 8)
            pltpu.make_async_copy(buf, out_hbm.at[cid, pl.ds(rp, CH)],
                                  gsems[j % 2]).wait()
        pltpu.sync_copy(shared.at[pl.ds(r, CH)], buf)
        pltpu.async_copy(buf, out_hbm.at[cid, pl.ds(r, CH)], gsems[j % 2])
    for j in range(nj - 2, nj):
        r = pl.multiple_of(row0 + j * CH, 8)
        pltpu.make_async_copy(rows[j % 2], out_hbm.at[cid, pl.ds(r, CH)],
                              gsems[j % 2]).wait()


def _prop_call(srcp3t, dstp3t, yb2):
    zfull = jnp.zeros((CH, DH), jnp.bfloat16)
    return pl.kernel(
        _prop_kernel,
        out_type=jax.ShapeDtypeStruct((NC, NP, DH), jnp.bfloat16),
        mesh=_sc_mesh(),
        compiler_params=pltpu.CompilerParams(use_tc_tiling_on_sc=False),
        scratch_types=[
            pltpu.VMEM((PNCH, CH), jnp.int32),
            pltpu.VMEM((PNCH, CH), jnp.int32),
            pltpu.VMEM((CH, DH), jnp.bfloat16),
            pltpu.VMEM((CH, DH), jnp.bfloat16),
            pltpu.VMEM((CH, DH), jnp.bfloat16),
            pltpu.VMEM((CH, DH), jnp.bfloat16),
            pltpu.VMEM((CH, DH), jnp.bfloat16),
            pltpu.VMEM_SHARED((NP, DH), jnp.bfloat16),
            pltpu.VMEM_SHARED((NP, DH), jnp.bfloat16),
            pltpu.SemaphoreType.DMA,
            pltpu.SemaphoreType.DMA,
            pltpu.SemaphoreType.DMA,
            pltpu.SemaphoreType.DMA,
            pltpu.SemaphoreType.DMA,
            pltpu.SemaphoreType.DMA,
            pltpu.SemaphoreType.DMA,
            pltpu.SemaphoreType.DMA,
            pltpu.SemaphoreType.DMA,
            pltpu.SemaphoreType.DMA,
        ],
    )(srcp3t, dstp3t, yb2, zfull)


def _prep_body(x_ref, degt_ref, yb2_ref, dinv_ref):
    deg = 1.0 + degt_ref[:, 0:1] + degt_ref[:, 1:2]
    dinv = lax.rsqrt(deg)
    dinv_ref[...] = dinv
    yb = (x_ref[...] * dinv).astype(jnp.bfloat16)
    yb2_ref[0] = yb[:, :DH]
    yb2_ref[1] = yb[:, DH:]


def _prep_call(x, degt):
    return pl.pallas_call(
        _prep_body,
        grid=(NB,),
        in_specs=[
            pl.BlockSpec((BLK, D), lambda i: (i, 0)),
            pl.BlockSpec((BLK, 2), lambda i: (i, 0)),
        ],
        out_specs=[
            pl.BlockSpec((NC, BLK, DH), lambda i: (0, i, 0)),
            pl.BlockSpec((BLK, 1), lambda i: (i, 0)),
        ],
        out_shape=[
            jax.ShapeDtypeStruct((NC, NP, DH), jnp.bfloat16),
            jax.ShapeDtypeStruct((N, 1), jnp.float32),
        ],
    )(x, degt)


def _finish_body(zp_ref, dinv_ref, x_ref, wexp_ref, bexp_ref,
                 w1_ref, b1_ref, w2_ref, b2_ref, out_ref,
                 zsc, g_sc, c_sc, wmp_sc, beta_sc):
    p = pl.program_id(0)
    i = pl.program_id(1)

    @pl.when(p == 0)
    def _phase0():
        by = jnp.concatenate([zp_ref[0], zp_ref[1]],
                             axis=1).astype(jnp.float32)
        dinv = dinv_ref[...]
        z = dinv * by + (dinv * dinv) * x_ref[...]
        zsc[pl.ds(i * BLK, BLK), :] = z
        gpart = lax.dot_general(z, z, (((0,), (0,)), ((), ())),
                                preferred_element_type=jnp.float32)
        cpart = jnp.sum(z, axis=0, keepdims=True)

        @pl.when(i == 0)
        def _init():
            g_sc[...] = gpart
            c_sc[...] = cpart

        @pl.when(i > 0)
        def _acc():
            g_sc[...] += gpart
            c_sc[...] += cpart

    @pl.when((p == 1) & (i == 0))
    def _scalars():
        wall = wexp_ref[...]
        ball = bexp_ref[...]
        s = jnp.zeros((D, D), jnp.float32)
        t = jnp.zeros((1, D), jnp.float32)
        for e in range(NE):
            we = wall[e]
            s = s + lax.dot_general(we, we, (((1,), (1,)), ((), ())),
                                    preferred_element_type=jnp.float32)
            t = t + lax.dot_general(ball[0, e:e + 1], we,
                                    (((1,), (1,)), ((), ())),
                                    preferred_element_type=jnp.float32)
        c1 = c_sc[...]
        wrs = jnp.sum(jnp.sum(wall, axis=0), axis=1)[None]
        bsum = jnp.sum(ball)
        bsq = jnp.sum(ball * ball)
        M = float(NE * N * D)
        sum_all = BN * (jnp.sum(c1 * wrs) + N * bsum)
        sumsq_all = BN * BN * (jnp.sum(g_sc[...] * s)
                               + 2.0 * jnp.sum(c1 * t) + N * bsq)
        gm = sum_all / M
        gs = jnp.sqrt(jnp.maximum(sumsq_all - M * gm * gm, 0.0) / (M - 1.0))
        inv = 1.0 / (gs + 1e-8)
        wm = jnp.sum(wall, axis=0) * (1.0 / NE)
        bm = jnp.sum(ball[0], axis=0, keepdims=True) * (1.0 / NE)
        wmp_sc[...] = (BN * inv) * wm
        beta_sc[...] = (BN * bm - gm) * inv

    @pl.when(p == 1)
    def _phase1():
        z = zsc[pl.ds(i * BLK, BLK), :]
        mm = lax.dot_general(z, wmp_sc[...], (((1,), (0,)), ((), ())),
                             preferred_element_type=jnp.float32)
        h = (x_ref[...] + mm + beta_sc[...]) * BN
        a1 = lax.dot_general(h, w1_ref[...], (((1,), (0,)), ((), ())),
                             preferred_element_type=jnp.float32)
        a1 = jnp.maximum(a1 + b1_ref[...], 0.0)
        ff = lax.dot_general(a1, w2_ref[...], (((1,), (0,)), ((), ())),
                             preferred_element_type=jnp.float32) + b2_ref[...]
        out_ref[...] = (h + ff) * BN


def _finish_call(zp, dinv, x, W_exp, b_exp, W1, b1, W2, b2):
    b_exp3 = b_exp[None]
    return pl.pallas_call(
        _finish_body,
        grid=(2, NB),
        in_specs=[
            pl.BlockSpec((NC, BLK, DH), lambda p, i: (0, i, 0)),
            pl.BlockSpec((BLK, 1), lambda p, i: (i, 0)),
            pl.BlockSpec((BLK, D), lambda p, i: (i, 0)),
            pl.BlockSpec((NE, D, D), lambda p, i: (0, 0, 0)),
            pl.BlockSpec((1, NE, D), lambda p, i: (0, 0, 0)),
            pl.BlockSpec((D, 2 * D), lambda p, i: (0, 0)),
            pl.BlockSpec((1, 2 * D), lambda p, i: (0, 0)),
            pl.BlockSpec((2 * D, D), lambda p, i: (0, 0)),
            pl.BlockSpec((1, D), lambda p, i: (0, 0)),
        ],
        out_specs=pl.BlockSpec((BLK, D), lambda p, i: (i, 0)),
        out_shape=jax.ShapeDtypeStruct((N, D), jnp.float32),
        scratch_shapes=[
            pltpu.VMEM((N, D), jnp.float32),
            pltpu.VMEM((D, D), jnp.float32),
            pltpu.VMEM((1, D), jnp.float32),
            pltpu.VMEM((D, D), jnp.float32),
            pltpu.VMEM((1, D), jnp.float32),
        ],
    )(zp, dinv, x, W_exp, b_exp3, W1, b1[None], W2, b2[None])


def kernel(x, edge_index, W_exp, b_exp, W1, b1, W2, b2):
    src = edge_index[0]
    dst = edge_index[1]
    pad = EP - E
    srcp = jnp.concatenate([src, jnp.zeros((pad,), jnp.int32)])
    dstp = jnp.concatenate(
        [dst, N + (jnp.arange(pad, dtype=jnp.int32) % (NP - N))])

    dstp3 = dstp.reshape(NW, DNCH, CH)
    srcp3t = srcp.reshape(NS, PNCH, CH)
    dstp3t = dstp.reshape(NS, PNCH, CH)

    degp = _deg_call(dstp3)
    degt = jnp.stack([degp[0, :N], degp[1, :N]], axis=1)
    yb2, dinv = _prep_call(x, degt)

    zp = _prop_call(srcp3t, dstp3t, yb2)
    return _finish_call(zp, dinv, x, W_exp, b_exp, W1, b1, W2, b2)
